# speedup vs baseline: 1.0040x; 1.0040x over previous
import jax
import jax.numpy as jnp
from jax.experimental import pallas as pl

N = 10000
E = 320000
IN_DIM = 128
HEADS = 8
HID = 16
NEG_SLOPE = 0.01

_BLK = 400


def _proj_kernel(x_ref, wp_ref, ws_ref, h_ref, sk_ref):
    x = x_ref[...]
    h_ref[...] = jnp.dot(x, wp_ref[...], preferred_element_type=jnp.float32)
    sk_ref[...] = jnp.dot(x, ws_ref[...], preferred_element_type=jnp.float32)


def kernel(x, edge_index, W_proj, att_e, W_skip):
    num_nodes = x.shape[0]
    wp_t = W_proj.T
    ws_t = W_skip.T
    h_flat, skip = pl.pallas_call(
        _proj_kernel,
        grid=(num_nodes // _BLK,),
        in_specs=[
            pl.BlockSpec((_BLK, IN_DIM), lambda i: (i, 0)),
            pl.BlockSpec((IN_DIM, HEADS * HID), lambda i: (0, 0)),
            pl.BlockSpec((IN_DIM, HEADS * HID), lambda i: (0, 0)),
        ],
        out_specs=[
            pl.BlockSpec((_BLK, HEADS * HID), lambda i: (i, 0)),
            pl.BlockSpec((_BLK, HEADS * HID), lambda i: (i, 0)),
        ],
        out_shape=[
            jax.ShapeDtypeStruct((num_nodes, HEADS * HID), jnp.float32),
            jax.ShapeDtypeStruct((num_nodes, HEADS * HID), jnp.float32),
        ],
    )(x, wp_t, ws_t)

    h = h_flat.reshape(num_nodes, HEADS, HID)
    src = edge_index[0]
    dst = edge_index[1]
    h_src = jnp.take(h, src, axis=0)
    h_dst = jnp.take(h, dst, axis=0)
    concat_h = jnp.concatenate([h_src, h_dst], axis=-1)
    e = jnp.sum(concat_h * att_e, axis=-1)
    e = jax.nn.leaky_relu(e, negative_slope=NEG_SLOPE)
    e_max = jax.ops.segment_max(e, dst, num_segments=num_nodes)
    e_exp = jnp.exp(e - jnp.take(e_max, dst, axis=0))
    denom = jax.ops.segment_sum(e_exp, dst, num_segments=num_nodes)
    alpha = e_exp / (jnp.take(denom, dst, axis=0) + 1e-16)
    h_prime = jax.ops.segment_sum(alpha[..., None] * h_src, dst, num_segments=num_nodes)
    h_prime = h_prime + skip.reshape(num_nodes, HEADS, HID)
    return h_prime.reshape(num_nodes, HEADS * HID)


# speedup vs baseline: 59.8404x; 59.5994x over previous
import functools

import jax
import jax.numpy as jnp
import numpy as np
from jax import lax
from jax.experimental import pallas as pl
from jax.experimental.pallas import tpu as pltpu
from jax.experimental.pallas import tpu_sc as plsc

N = 10000
E = 320000
IN_DIM = 128
HEADS = 8
HID = 16
HD = HEADS * HID
AW = HD + 16
NEG_SLOPE = 0.01

NC = 2
NS = 16
NW = NC * NS
LANES = 16

_TC_BLK = 400


def _pre_kernel(x_ref, wp_ref, ac_ref, ws_ref, h_ref, scat_ref, skip_ref):
    x = x_ref[...]
    h = jnp.dot(x, wp_ref[...], preferred_element_type=jnp.float32)
    h_ref[...] = h
    scat_ref[...] = jnp.dot(h, ac_ref[...], preferred_element_type=jnp.float32)
    skip_ref[...] = jnp.dot(x, ws_ref[...], preferred_element_type=jnp.float32)


def _merge_kernel(a0_ref, a1_ref, skip_ref, e8_ref, out_ref):
    a = a0_ref[0] + a1_ref[0]
    u = a[:, :HD]
    p = a[:, HD:HD + HEADS]
    r = 1.0 / (p + 1e-16)
    r128 = jnp.dot(r, e8_ref[...], preferred_element_type=jnp.float32)
    out_ref[...] = u * r128 + skip_ref[...]


def _make_sc_kernel(n_nodes, n_edges, blk):
    epw = n_edges // NW
    n_chunks = epw // blk
    rows_per_tile = n_nodes // NS
    n_win = pl.cdiv(n_nodes, blk)
    mesh = plsc.VectorSubcoreMesh(core_axis_name="c", subcore_axis_name="s")

    dnums = lax.GatherDimensionNumbers(
        offset_dims=(), collapsed_slice_dims=(0,), start_index_map=(0,))

    def lane_gather(v, idx):
        return lax.gather(v, idx.reshape(LANES, 1), dnums, (1,),
                          mode=lax.GatherScatterMode.PROMISE_IN_BOUNDS)

    @functools.partial(
        pl.kernel,
        out_type=jax.ShapeDtypeStruct((NC, n_nodes, AW), jnp.float32),
        mesh=mesh,
        compiler_params=pltpu.CompilerParams(use_tc_tiling_on_sc=False),
        scratch_types=[
            pltpu.VMEM_SHARED((n_nodes, AW), jnp.float32),
            pltpu.VMEM((blk,), jnp.int32),
            pltpu.VMEM((blk,), jnp.int32),
            pltpu.VMEM((blk, LANES), jnp.float32),
            pltpu.VMEM((blk, LANES), jnp.float32),
            pltpu.VMEM((blk, HD), jnp.float32),
            pltpu.VMEM((blk,), jnp.int32),
            pltpu.VMEM((blk,), jnp.int32),
            pltpu.VMEM((blk, LANES), jnp.float32),
            pltpu.VMEM((blk, LANES), jnp.float32),
            pltpu.VMEM((blk, HD), jnp.float32),
            pltpu.VMEM((blk, AW), jnp.float32),
            pltpu.SemaphoreType.DMA,
            pltpu.SemaphoreType.DMA,
            pltpu.SemaphoreType.DMA,
            pltpu.SemaphoreType.DMA,
        ],
    )
    def sc_edge_pass(scat_hbm, h_hbm, src_hbm, dst_hbm, out_hbm,
                     acc, srcb0, dstb0, sbuf0, dbuf0, hbuf0,
                     srcb1, dstb1, sbuf1, dbuf1, hbuf1, stage,
                     sem_i0, sem_i1, sem_g0, sem_g1):
        cid = lax.axis_index("c")
        sid = lax.axis_index("s")
        wid = cid * NS + sid

        lane_iota = lax.iota(jnp.int32, LANES)
        shift_idx = (lane_iota & 7) + 8
        head_idx = [jnp.full((LANES,), hh, jnp.int32) for hh in range(HEADS)]
        zero16 = jnp.zeros((LANES,), jnp.float32)

        sets = ((srcb0, dstb0, sbuf0, dbuf0, hbuf0, sem_i0, sem_g0),
                (srcb1, dstb1, sbuf1, dbuf1, hbuf1, sem_i1, sem_g1))

        @pl.loop(0, blk)
        def _(j):
            for col in range(AW // LANES):
                stage[j, pl.ds(col * LANES, LANES)] = zero16

        for k in range(pl.cdiv(n_win, NS)):
            w = sid + NS * k

            @pl.when(w < n_win)
            def _():
                pltpu.sync_copy(stage, acc.at[pl.ds(w * blk, blk)])

        plsc.subcore_barrier()

        def issue_idx(c, bs):
            base = wid * epw + c * blk
            pltpu.async_copy(src_hbm.at[pl.ds(base, blk)], bs[0], bs[5])
            pltpu.async_copy(dst_hbm.at[pl.ds(base, blk)], bs[1], bs[5])

        def wait_idx(c, bs):
            base = wid * epw + c * blk
            pltpu.make_async_copy(src_hbm.at[pl.ds(base, blk)], bs[0], bs[5]).wait()
            pltpu.make_async_copy(dst_hbm.at[pl.ds(base, blk)], bs[1], bs[5]).wait()

        def issue_gather(bs):
            pltpu.async_copy(scat_hbm.at[bs[0]], bs[2], bs[6])
            pltpu.async_copy(scat_hbm.at[bs[1]], bs[3], bs[6])
            pltpu.async_copy(h_hbm.at[bs[0]], bs[4], bs[6])

        def wait_gather(bs):
            pltpu.make_async_copy(scat_hbm.at[bs[0]], bs[2], bs[6]).wait()
            pltpu.make_async_copy(scat_hbm.at[bs[1]], bs[3], bs[6]).wait()
            pltpu.make_async_copy(h_hbm.at[bs[0]], bs[4], bs[6]).wait()

        def compute(bs):
            sb, db, hb = bs[2], bs[3], bs[4]

            @pl.loop(0, blk)
            def _(j):
                rs = sb[j, :]
                rd = db[j, :]
                sd = lane_gather(rd, shift_idx)
                e = rs + sd
                e = jnp.where(e >= 0.0, e, e * NEG_SLOPE)
                p = jnp.exp(e)
                stage[j, pl.ds(HD, LANES)] = p
                for hh in range(HEADS):
                    pv = lane_gather(p, head_idx[hh])
                    hv = hb[j, pl.ds(hh * LANES, LANES)]
                    stage[j, pl.ds(hh * LANES, LANES)] = hv * pv

        def scatter(bs):
            pltpu.sync_copy(stage, acc.at[bs[1]], add=True)

        def body(cc, k):
            bs = sets[k]
            other = sets[1 - k]

            @pl.when(cc + 1 < n_chunks)
            def _():
                wait_idx(cc + 1, other)
                issue_gather(other)

            wait_gather(bs)
            compute(bs)
            scatter(bs)

            @pl.when(cc + 2 < n_chunks)
            def _():
                issue_idx(cc + 2, bs)

        assert n_chunks % 2 == 1
        issue_idx(0, sets[0])
        wait_idx(0, sets[0])
        issue_gather(sets[0])
        issue_idx(1, sets[1])

        @pl.loop(0, n_chunks - 1, step=2)
        def _(c):
            body(c, 0)
            body(c + 1, 1)

        body(jnp.int32(n_chunks - 1), 0)

        plsc.subcore_barrier()

        r0 = sid * rows_per_tile
        pltpu.sync_copy(acc.at[pl.ds(r0, rows_per_tile)],
                        out_hbm.at[cid].at[pl.ds(r0, rows_per_tile)])

    return sc_edge_pass


_SC_BLK = 80


def kernel(x, edge_index, W_proj, att_e, W_skip):
    num_nodes = x.shape[0]

    wp_t = W_proj.T
    ws_t = W_skip.T
    a_src = att_e[0, :, :HID]
    a_dst = att_e[0, :, HID:]
    eye8 = jnp.eye(HEADS, dtype=jnp.float32)
    a_cat = jnp.concatenate(
        [
            (eye8[:, None, :] * a_src[:, :, None]).reshape(HD, HEADS),
            (eye8[:, None, :] * a_dst[:, :, None]).reshape(HD, HEADS),
        ],
        axis=1,
    )

    h, scat, skip = pl.pallas_call(
        _pre_kernel,
        grid=(num_nodes // _TC_BLK,),
        in_specs=[
            pl.BlockSpec((_TC_BLK, IN_DIM), lambda i: (i, 0)),
            pl.BlockSpec((IN_DIM, HD), lambda i: (0, 0)),
            pl.BlockSpec((HD, 2 * HEADS), lambda i: (0, 0)),
            pl.BlockSpec((IN_DIM, HD), lambda i: (0, 0)),
        ],
        out_specs=[
            pl.BlockSpec((_TC_BLK, HD), lambda i: (i, 0)),
            pl.BlockSpec((_TC_BLK, 2 * HEADS), lambda i: (i, 0)),
            pl.BlockSpec((_TC_BLK, HD), lambda i: (i, 0)),
        ],
        out_shape=[
            jax.ShapeDtypeStruct((num_nodes, HD), jnp.float32),
            jax.ShapeDtypeStruct((num_nodes, 2 * HEADS), jnp.float32),
            jax.ShapeDtypeStruct((num_nodes, HD), jnp.float32),
        ],
    )(x, wp_t, a_cat, ws_t)

    sc = _make_sc_kernel(num_nodes, E, _SC_BLK)
    acc = sc(scat, h, edge_index[0], edge_index[1])

    e8 = jnp.asarray(np.repeat(np.eye(HEADS, dtype=np.float32), HID, axis=1))

    out = pl.pallas_call(
        _merge_kernel,
        grid=(num_nodes // _TC_BLK,),
        in_specs=[
            pl.BlockSpec((1, _TC_BLK, AW), lambda i: (0, i, 0)),
            pl.BlockSpec((1, _TC_BLK, AW), lambda i: (1, i, 0)),
            pl.BlockSpec((_TC_BLK, HD), lambda i: (i, 0)),
            pl.BlockSpec((HEADS, HD), lambda i: (0, 0)),
        ],
        out_specs=pl.BlockSpec((_TC_BLK, HD), lambda i: (i, 0)),
        out_shape=jax.ShapeDtypeStruct((num_nodes, HD), jnp.float32),
    )(acc, acc, skip, e8)

    return out


# speedup vs baseline: 146.8998x; 2.4549x over previous
import functools

import jax
import jax.numpy as jnp
import numpy as np
from jax import lax
from jax.experimental import pallas as pl
from jax.experimental.pallas import tpu as pltpu
from jax.experimental.pallas import tpu_sc as plsc

N = 10000
E = 320000
IN_DIM = 128
HEADS = 8
HID = 16
HD = HEADS * HID
AW = HD + 16
NEG_SLOPE = 0.01

NC = 2
NS = 16
NW = NC * NS
LANES = 16

_TC_BLK = 400


def _pre_kernel(x_ref, wp_ref, ac_ref, ws_ref, h_ref, scat_ref, skip_ref):
    x = x_ref[...]
    h = jnp.dot(x, wp_ref[...], preferred_element_type=jnp.float32)
    h_ref[...] = h
    scat_ref[...] = jnp.dot(h, ac_ref[...], preferred_element_type=jnp.float32)
    skip_ref[...] = jnp.dot(x, ws_ref[...], preferred_element_type=jnp.float32)


def _merge_kernel(a0_ref, a1_ref, skip_ref, e8_ref, out_ref):
    a = a0_ref[0] + a1_ref[0]
    u = a[:, :HD]
    p = a[:, HD:HD + HEADS]
    r = 1.0 / (p + 1e-16)
    r128 = jnp.dot(r, e8_ref[...], preferred_element_type=jnp.float32)
    out_ref[...] = u * r128 + skip_ref[...]


def _make_sc_kernel(n_nodes, n_edges, blk):
    epw = n_edges // NW
    n_chunks = epw // blk
    rows_per_tile = n_nodes // NS
    n_win = pl.cdiv(n_nodes, blk)
    mesh = plsc.VectorSubcoreMesh(core_axis_name="c", subcore_axis_name="s")

    dnums = lax.GatherDimensionNumbers(
        offset_dims=(), collapsed_slice_dims=(0,), start_index_map=(0,))

    def lane_gather(v, idx):
        return lax.gather(v, idx.reshape(LANES, 1), dnums, (1,),
                          mode=lax.GatherScatterMode.PROMISE_IN_BOUNDS)

    @functools.partial(
        pl.kernel,
        out_type=jax.ShapeDtypeStruct((NC, n_nodes, AW), jnp.float32),
        mesh=mesh,
        compiler_params=pltpu.CompilerParams(use_tc_tiling_on_sc=False),
        scratch_types=[
            pltpu.VMEM_SHARED((n_nodes, AW), jnp.float32),
            pltpu.VMEM((blk,), jnp.int32),
            pltpu.VMEM((blk,), jnp.int32),
            pltpu.VMEM((blk, LANES), jnp.float32),
            pltpu.VMEM((blk, LANES), jnp.float32),
            pltpu.VMEM((blk, HD), jnp.float32),
            pltpu.VMEM((blk,), jnp.int32),
            pltpu.VMEM((blk,), jnp.int32),
            pltpu.VMEM((blk, LANES), jnp.float32),
            pltpu.VMEM((blk, LANES), jnp.float32),
            pltpu.VMEM((blk, HD), jnp.float32),
            pltpu.VMEM((blk, AW), jnp.float32),
            pltpu.SemaphoreType.DMA,
            pltpu.SemaphoreType.DMA,
            pltpu.SemaphoreType.DMA,
            pltpu.SemaphoreType.DMA,
        ],
    )
    def sc_edge_pass(scat_hbm, h_hbm, src_hbm, dst_hbm, out_hbm,
                     acc, srcb0, dstb0, sbuf0, dbuf0, hbuf0,
                     srcb1, dstb1, sbuf1, dbuf1, hbuf1, stage,
                     sem_i0, sem_i1, sem_g0, sem_g1):
        cid = lax.axis_index("c")
        sid = lax.axis_index("s")
        wid = cid * NS + sid

        lane_iota = lax.iota(jnp.int32, LANES)
        shift_idx = (lane_iota & 7) + 8
        head_idx = [jnp.full((LANES,), hh, jnp.int32) for hh in range(HEADS)]
        zero16 = jnp.zeros((LANES,), jnp.float32)

        sets = ((srcb0, dstb0, sbuf0, dbuf0, hbuf0, sem_i0, sem_g0),
                (srcb1, dstb1, sbuf1, dbuf1, hbuf1, sem_i1, sem_g1))

        @pl.loop(0, blk)
        def _(j):
            for col in range(AW // LANES):
                stage[j, pl.ds(col * LANES, LANES)] = zero16

        for k in range(pl.cdiv(n_win, NS)):
            w = sid + NS * k

            @pl.when(w < n_win)
            def _():
                pltpu.sync_copy(stage, acc.at[pl.ds(w * blk, blk)])

        plsc.subcore_barrier()

        def issue_idx(c, bs):
            base = wid * epw + c * blk
            pltpu.async_copy(src_hbm.at[pl.ds(base, blk)], bs[0], bs[5])
            pltpu.async_copy(dst_hbm.at[pl.ds(base, blk)], bs[1], bs[5])

        def wait_idx(c, bs):
            base = wid * epw + c * blk
            pltpu.make_async_copy(src_hbm.at[pl.ds(base, blk)], bs[0], bs[5]).wait()
            pltpu.make_async_copy(dst_hbm.at[pl.ds(base, blk)], bs[1], bs[5]).wait()

        def issue_gather(bs):
            pltpu.async_copy(scat_hbm.at[bs[0]], bs[2], bs[6])
            pltpu.async_copy(scat_hbm.at[bs[1]], bs[3], bs[6])
            pltpu.async_copy(h_hbm.at[bs[0]], bs[4], bs[6])

        def wait_gather(bs):
            pltpu.make_async_copy(scat_hbm.at[bs[0]], bs[2], bs[6]).wait()
            pltpu.make_async_copy(scat_hbm.at[bs[1]], bs[3], bs[6]).wait()
            pltpu.make_async_copy(h_hbm.at[bs[0]], bs[4], bs[6]).wait()

        def compute(bs):
            sb, db, hb = bs[2], bs[3], bs[4]

            @plsc.parallel_loop(0, blk, unroll=2)
            def _(j):
                rs = sb[j, :]
                rd = db[j, :]
                sd = lane_gather(rd, shift_idx)
                e = rs + sd
                e = jnp.where(e >= 0.0, e, e * NEG_SLOPE)
                p = jnp.exp(e)
                stage[j, pl.ds(HD, LANES)] = p
                for hh in range(HEADS):
                    pv = lane_gather(p, head_idx[hh])
                    hv = hb[j, pl.ds(hh * LANES, LANES)]
                    stage[j, pl.ds(hh * LANES, LANES)] = hv * pv

        def scatter(bs):
            pltpu.sync_copy(stage, acc.at[bs[1]], add=True)

        def body(cc, k):
            bs = sets[k]
            other = sets[1 - k]

            @pl.when(cc + 1 < n_chunks)
            def _():
                wait_idx(cc + 1, other)
                issue_gather(other)

            wait_gather(bs)
            compute(bs)
            scatter(bs)

            @pl.when(cc + 2 < n_chunks)
            def _():
                issue_idx(cc + 2, bs)

        assert n_chunks % 2 == 1
        issue_idx(0, sets[0])
        wait_idx(0, sets[0])
        issue_gather(sets[0])
        issue_idx(1, sets[1])

        @pl.loop(0, n_chunks - 1, step=2)
        def _(c):
            body(c, 0)
            body(c + 1, 1)

        body(jnp.int32(n_chunks - 1), 0)

        plsc.subcore_barrier()

        r0 = sid * rows_per_tile
        pltpu.sync_copy(acc.at[pl.ds(r0, rows_per_tile)],
                        out_hbm.at[cid].at[pl.ds(r0, rows_per_tile)])

    return sc_edge_pass


_SC_BLK = 80


def kernel(x, edge_index, W_proj, att_e, W_skip):
    num_nodes = x.shape[0]

    wp_t = W_proj.T
    ws_t = W_skip.T
    a_src = att_e[0, :, :HID]
    a_dst = att_e[0, :, HID:]
    eye8 = jnp.eye(HEADS, dtype=jnp.float32)
    a_cat = jnp.concatenate(
        [
            (eye8[:, None, :] * a_src[:, :, None]).reshape(HD, HEADS),
            (eye8[:, None, :] * a_dst[:, :, None]).reshape(HD, HEADS),
        ],
        axis=1,
    )

    h, scat, skip = pl.pallas_call(
        _pre_kernel,
        grid=(num_nodes // _TC_BLK,),
        in_specs=[
            pl.BlockSpec((_TC_BLK, IN_DIM), lambda i: (i, 0)),
            pl.BlockSpec((IN_DIM, HD), lambda i: (0, 0)),
            pl.BlockSpec((HD, 2 * HEADS), lambda i: (0, 0)),
            pl.BlockSpec((IN_DIM, HD), lambda i: (0, 0)),
        ],
        out_specs=[
            pl.BlockSpec((_TC_BLK, HD), lambda i: (i, 0)),
            pl.BlockSpec((_TC_BLK, 2 * HEADS), lambda i: (i, 0)),
            pl.BlockSpec((_TC_BLK, HD), lambda i: (i, 0)),
        ],
        out_shape=[
            jax.ShapeDtypeStruct((num_nodes, HD), jnp.float32),
            jax.ShapeDtypeStruct((num_nodes, 2 * HEADS), jnp.float32),
            jax.ShapeDtypeStruct((num_nodes, HD), jnp.float32),
        ],
    )(x, wp_t, a_cat, ws_t)

    sc = _make_sc_kernel(num_nodes, E, _SC_BLK)
    acc = sc(scat, h, edge_index[0], edge_index[1])

    e8 = jnp.asarray(np.repeat(np.eye(HEADS, dtype=np.float32), HID, axis=1))

    out = pl.pallas_call(
        _merge_kernel,
        grid=(num_nodes // _TC_BLK,),
        in_specs=[
            pl.BlockSpec((1, _TC_BLK, AW), lambda i: (0, i, 0)),
            pl.BlockSpec((1, _TC_BLK, AW), lambda i: (1, i, 0)),
            pl.BlockSpec((_TC_BLK, HD), lambda i: (i, 0)),
            pl.BlockSpec((HEADS, HD), lambda i: (0, 0)),
        ],
        out_specs=pl.BlockSpec((_TC_BLK, HD), lambda i: (i, 0)),
        out_shape=jax.ShapeDtypeStruct((num_nodes, HD), jnp.float32),
    )(acc, acc, skip, e8)

    return out


# speedup vs baseline: 148.2441x; 1.0092x over previous
import functools

import jax
import jax.numpy as jnp
import numpy as np
from jax import lax
from jax.experimental import pallas as pl
from jax.experimental.pallas import tpu as pltpu
from jax.experimental.pallas import tpu_sc as plsc

N = 10000
E = 320000
IN_DIM = 128
HEADS = 8
HID = 16
HD = HEADS * HID
AW = HD + 16
NEG_SLOPE = 0.01

NC = 2
NS = 16
NW = NC * NS
LANES = 16

_TC_BLK = 400


def _pre_kernel(x_ref, wp_ref, ac_ref, ws_ref, h_ref, scat_ref, skip_ref):
    x = x_ref[...]
    h = jnp.dot(x, wp_ref[...], preferred_element_type=jnp.float32)
    h_ref[...] = h
    scat_ref[...] = jnp.dot(h, ac_ref[...], preferred_element_type=jnp.float32)
    skip_ref[...] = jnp.dot(x, ws_ref[...], preferred_element_type=jnp.float32)


def _merge_kernel(a0_ref, a1_ref, skip_ref, e8_ref, out_ref):
    a = a0_ref[0] + a1_ref[0]
    u = a[:, :HD]
    p = a[:, HD:HD + HEADS]
    r = 1.0 / (p + 1e-16)
    r128 = jnp.dot(r, e8_ref[...], preferred_element_type=jnp.float32)
    out_ref[...] = u * r128 + skip_ref[...]


def _make_sc_kernel(n_nodes, n_edges, blk):
    epw = n_edges // NW
    n_chunks = epw // blk
    rows_per_tile = n_nodes // NS
    n_win = pl.cdiv(n_nodes, blk)
    mesh = plsc.VectorSubcoreMesh(core_axis_name="c", subcore_axis_name="s")

    dnums = lax.GatherDimensionNumbers(
        offset_dims=(), collapsed_slice_dims=(0,), start_index_map=(0,))

    def lane_gather(v, idx):
        return lax.gather(v, idx.reshape(LANES, 1), dnums, (1,),
                          mode=lax.GatherScatterMode.PROMISE_IN_BOUNDS)

    @functools.partial(
        pl.kernel,
        out_type=jax.ShapeDtypeStruct((NC, n_nodes, AW), jnp.float32),
        mesh=mesh,
        compiler_params=pltpu.CompilerParams(use_tc_tiling_on_sc=False),
        scratch_types=[
            pltpu.VMEM_SHARED((n_nodes, AW), jnp.float32),
            pltpu.VMEM((blk,), jnp.int32),
            pltpu.VMEM((blk,), jnp.int32),
            pltpu.VMEM((blk, LANES), jnp.float32),
            pltpu.VMEM((blk, LANES), jnp.float32),
            pltpu.VMEM((blk, HD), jnp.float32),
            pltpu.VMEM((blk,), jnp.int32),
            pltpu.VMEM((blk,), jnp.int32),
            pltpu.VMEM((blk, LANES), jnp.float32),
            pltpu.VMEM((blk, LANES), jnp.float32),
            pltpu.VMEM((blk, HD), jnp.float32),
            pltpu.VMEM((blk, AW), jnp.float32),
            pltpu.SemaphoreType.DMA,
            pltpu.SemaphoreType.DMA,
            pltpu.SemaphoreType.DMA,
            pltpu.SemaphoreType.DMA,
        ],
    )
    def sc_edge_pass(scat_hbm, h_hbm, src_hbm, dst_hbm, out_hbm,
                     acc, srcb0, dstb0, sbuf0, dbuf0, hbuf0,
                     srcb1, dstb1, sbuf1, dbuf1, hbuf1, stage,
                     sem_i0, sem_i1, sem_g0, sem_g1):
        cid = lax.axis_index("c")
        sid = lax.axis_index("s")
        wid = cid * NS + sid

        lane_iota = lax.iota(jnp.int32, LANES)
        shift_idx = (lane_iota & 7) + 8
        head_idx = [jnp.full((LANES,), hh, jnp.int32) for hh in range(HEADS)]
        zero16 = jnp.zeros((LANES,), jnp.float32)

        sets = ((srcb0, dstb0, sbuf0, dbuf0, hbuf0, sem_i0, sem_g0),
                (srcb1, dstb1, sbuf1, dbuf1, hbuf1, sem_i1, sem_g1))

        @pl.loop(0, blk)
        def _(j):
            for col in range(AW // LANES):
                stage[j, pl.ds(col * LANES, LANES)] = zero16

        for k in range(pl.cdiv(n_win, NS)):
            w = sid + NS * k

            @pl.when(w < n_win)
            def _():
                pltpu.sync_copy(stage, acc.at[pl.ds(w * blk, blk)])

        plsc.subcore_barrier()

        def issue_idx(c, bs):
            base = wid * epw + c * blk
            pltpu.async_copy(src_hbm.at[pl.ds(base, blk)], bs[0], bs[5])
            pltpu.async_copy(dst_hbm.at[pl.ds(base, blk)], bs[1], bs[5])

        def wait_idx(c, bs):
            base = wid * epw + c * blk
            pltpu.make_async_copy(src_hbm.at[pl.ds(base, blk)], bs[0], bs[5]).wait()
            pltpu.make_async_copy(dst_hbm.at[pl.ds(base, blk)], bs[1], bs[5]).wait()

        def issue_gather(bs):
            pltpu.async_copy(scat_hbm.at[bs[0]], bs[2], bs[6])
            pltpu.async_copy(scat_hbm.at[bs[1]], bs[3], bs[6])
            pltpu.async_copy(h_hbm.at[bs[0]], bs[4], bs[6])

        def wait_gather(bs):
            pltpu.make_async_copy(scat_hbm.at[bs[0]], bs[2], bs[6]).wait()
            pltpu.make_async_copy(scat_hbm.at[bs[1]], bs[3], bs[6]).wait()
            pltpu.make_async_copy(h_hbm.at[bs[0]], bs[4], bs[6]).wait()

        def compute(bs):
            sb, db, hb = bs[2], bs[3], bs[4]

            @plsc.parallel_loop(0, blk, unroll=4)
            def _(j):
                rs = sb[j, :]
                rd = db[j, :]
                sd = lane_gather(rd, shift_idx)
                e = rs + sd
                e = jnp.where(e >= 0.0, e, e * NEG_SLOPE)
                p = jnp.exp(e)
                stage[j, pl.ds(HD, LANES)] = p
                for hh in range(HEADS):
                    pv = lane_gather(p, head_idx[hh])
                    hv = hb[j, pl.ds(hh * LANES, LANES)]
                    stage[j, pl.ds(hh * LANES, LANES)] = hv * pv

        def scatter(bs):
            pltpu.sync_copy(stage, acc.at[bs[1]], add=True)

        def body(cc, k):
            bs = sets[k]
            other = sets[1 - k]

            @pl.when(cc + 1 < n_chunks)
            def _():
                wait_idx(cc + 1, other)
                issue_gather(other)

            wait_gather(bs)
            compute(bs)
            scatter(bs)

            @pl.when(cc + 2 < n_chunks)
            def _():
                issue_idx(cc + 2, bs)

        assert n_chunks % 2 == 1
        issue_idx(0, sets[0])
        wait_idx(0, sets[0])
        issue_gather(sets[0])
        issue_idx(1, sets[1])

        @pl.loop(0, n_chunks - 1, step=2)
        def _(c):
            body(c, 0)
            body(c + 1, 1)

        body(jnp.int32(n_chunks - 1), 0)

        plsc.subcore_barrier()

        r0 = sid * rows_per_tile
        pltpu.sync_copy(acc.at[pl.ds(r0, rows_per_tile)],
                        out_hbm.at[cid].at[pl.ds(r0, rows_per_tile)])

    return sc_edge_pass


_SC_BLK = 80


def kernel(x, edge_index, W_proj, att_e, W_skip):
    num_nodes = x.shape[0]

    wp_t = W_proj.T
    ws_t = W_skip.T
    a_src = att_e[0, :, :HID]
    a_dst = att_e[0, :, HID:]
    eye8 = jnp.eye(HEADS, dtype=jnp.float32)
    a_cat = jnp.concatenate(
        [
            (eye8[:, None, :] * a_src[:, :, None]).reshape(HD, HEADS),
            (eye8[:, None, :] * a_dst[:, :, None]).reshape(HD, HEADS),
        ],
        axis=1,
    )

    h, scat, skip = pl.pallas_call(
        _pre_kernel,
        grid=(num_nodes // _TC_BLK,),
        in_specs=[
            pl.BlockSpec((_TC_BLK, IN_DIM), lambda i: (i, 0)),
            pl.BlockSpec((IN_DIM, HD), lambda i: (0, 0)),
            pl.BlockSpec((HD, 2 * HEADS), lambda i: (0, 0)),
            pl.BlockSpec((IN_DIM, HD), lambda i: (0, 0)),
        ],
        out_specs=[
            pl.BlockSpec((_TC_BLK, HD), lambda i: (i, 0)),
            pl.BlockSpec((_TC_BLK, 2 * HEADS), lambda i: (i, 0)),
            pl.BlockSpec((_TC_BLK, HD), lambda i: (i, 0)),
        ],
        out_shape=[
            jax.ShapeDtypeStruct((num_nodes, HD), jnp.float32),
            jax.ShapeDtypeStruct((num_nodes, 2 * HEADS), jnp.float32),
            jax.ShapeDtypeStruct((num_nodes, HD), jnp.float32),
        ],
    )(x, wp_t, a_cat, ws_t)

    sc = _make_sc_kernel(num_nodes, E, _SC_BLK)
    acc = sc(scat, h, edge_index[0], edge_index[1])

    e8 = jnp.asarray(np.repeat(np.eye(HEADS, dtype=np.float32), HID, axis=1))

    out = pl.pallas_call(
        _merge_kernel,
        grid=(num_nodes // _TC_BLK,),
        in_specs=[
            pl.BlockSpec((1, _TC_BLK, AW), lambda i: (0, i, 0)),
            pl.BlockSpec((1, _TC_BLK, AW), lambda i: (1, i, 0)),
            pl.BlockSpec((_TC_BLK, HD), lambda i: (i, 0)),
            pl.BlockSpec((HEADS, HD), lambda i: (0, 0)),
        ],
        out_specs=pl.BlockSpec((_TC_BLK, HD), lambda i: (i, 0)),
        out_shape=jax.ShapeDtypeStruct((num_nodes, HD), jnp.float32),
    )(acc, acc, skip, e8)

    return out
